# baseline (speedup 1.0000x reference)
import functools

import jax
import jax.numpy as jnp
from jax import lax
from jax.experimental import pallas as pl
from jax.experimental.pallas import tpu as pltpu
from jax.experimental.pallas import tpu_sc as plsc

NC, NS, LANES = 2, 16, 16
NW = NC * NS

B, F_, L_, V_, D_ = 1024, 26, 20, 1000, 64
N = B * F_ * L_
SUB = 128
NSUB = 8
CHUNK = SUB * NSUB
NCHUNK = N // CHUNK
PERIOD = F_ * L_
OFF_LEN = PERIOD + CHUNK
HALF = NSUB // 2

PIECES = 2
CPP = NCHUNK // PIECES


def _piece_body(piece, idx_hbm, table_hbm, off_hbm, out_hbm, idx_v, rows_v,
                off_v, gsem0, gsem1, ssem0, ssem1):
    wid = lax.axis_index("s") * NC + lax.axis_index("c")
    gsems = (gsem0, gsem1)
    ssems = (ssem0, ssem1)
    pltpu.sync_copy(off_hbm, off_v)
    n_mine = jnp.where(wid < CPP % NW, CPP // NW + 1, CPP // NW)
    piece_base = piece * CPP * CHUNK

    def make_store(local_base, h):
        return pltpu.make_async_copy(
            rows_v.at[h],
            out_hbm.at[pl.ds(local_base + h * (CHUNK // 2), CHUNK // 2)],
            ssems[h],
        )

    def chunk(g, first):
        base = pl.multiple_of(piece_base + (g * NW + wid) * CHUNK, CHUNK)
        local_base = pl.multiple_of(base - piece_base, CHUNK)
        phase = base % PERIOD
        row0 = pl.multiple_of(base // SUB, NSUB)
        pltpu.sync_copy(idx_hbm.at[pl.ds(row0, NSUB)], idx_v)
        for j in range(NSUB):
            for k in range(SUB // LANES):
                off = off_v[pl.ds(phase + j * SUB + k * LANES, LANES)]
                idx_v[j, pl.ds(k * LANES, LANES)] = (
                    idx_v[j, pl.ds(k * LANES, LANES)] + off
                )
        for h in (0, 1):
            gathers = [
                pltpu.make_async_copy(
                    table_hbm.at[idx_v.at[h * HALF + j]],
                    rows_v.at[h, pl.ds(j * SUB, SUB)],
                    gsems[h],
                )
                for j in range(HALF)
            ]
            @pl.when(jnp.logical_not(first))
            def _():
                make_store(local_base, h).wait()
            for c in gathers:
                c.start()
            for c in gathers:
                c.wait()
            make_store(local_base, h).start()
        return jnp.bool_(False)

    lax.fori_loop(0, n_mine, chunk, jnp.bool_(True))
    for h in (0, 1):
        make_store(0, h).wait()


@jax.jit
def kernel(indices, tables):
    flat_tables = tables.reshape(F_ * V_, D_)
    idx2d = indices.reshape(N // SUB, SUB)
    off_lut = (jnp.arange(OFF_LEN, dtype=jnp.int32) // L_ % F_) * V_
    mesh = plsc.VectorSubcoreMesh(
        core_axis_name="c", subcore_axis_name="s", num_cores=NC, num_subcores=NS
    )
    pieces = []
    for p in range(PIECES):
        out_p = pl.kernel(
            functools.partial(_piece_body, p),
            out_type=jax.ShapeDtypeStruct((CPP * CHUNK, D_), jnp.float32),
            mesh=mesh,
            scratch_types=[
                pltpu.VMEM((NSUB, SUB), jnp.int32),
                pltpu.VMEM((2, CHUNK // 2, D_), jnp.float32),
                pltpu.VMEM((OFF_LEN,), jnp.int32),
                pltpu.SemaphoreType.DMA,
                pltpu.SemaphoreType.DMA,
                pltpu.SemaphoreType.DMA,
                pltpu.SemaphoreType.DMA,
            ],
            compiler_params=pltpu.CompilerParams(use_tc_tiling_on_sc=False),
            name=f"gather_piece_{p}",
        )(idx2d, flat_tables, off_lut)
        pieces.append(out_p.reshape(B // PIECES, F_ * L_ * D_))
    return jnp.concatenate(pieces, axis=0)


# speedup vs baseline: 1.2310x; 1.2310x over previous
import jax
import jax.numpy as jnp
from jax import lax
from jax.experimental import pallas as pl
from jax.experimental.pallas import tpu as pltpu
from jax.experimental.pallas import tpu_sc as plsc

NC, NS, LANES = 2, 16, 16
NW = NC * NS

B, F_, L_, V_, D_ = 1024, 26, 20, 1000, 64
N = B * F_ * L_
SUB = 128
NSUB = 8
CHUNK = SUB * NSUB
NCHUNK = N // CHUNK
PERIOD = F_ * L_
OFF_LEN = PERIOD + CHUNK
HALF = NSUB // 2


def _body(idx_hbm, table_hbm, off_hbm, out_hbm, idx_v, rows_v, off_v,
          gsem0, gsem1, ssem0, ssem1, isem):
    wid = lax.axis_index("s") * NC + lax.axis_index("c")
    gsems = (gsem0, gsem1)
    ssems = (ssem0, ssem1)
    pltpu.sync_copy(off_hbm, off_v)
    n_mine = jnp.where(wid < NCHUNK % NW, NCHUNK // NW + 1, NCHUNK // NW)

    def chunk_base(g):
        return pl.multiple_of((g * NW + wid) * CHUNK, CHUNK)

    def idx_load(g, slot):
        base = chunk_base(jnp.minimum(g, n_mine - 1))
        row0 = pl.multiple_of(base // SUB, NSUB)
        return pltpu.make_async_copy(
            idx_hbm.at[pl.ds(row0, NSUB)], idx_v.at[slot], isem
        )

    def make_store(base, h):
        return pltpu.make_async_copy(
            rows_v.at[h],
            out_hbm.at[pl.ds(base + h * (CHUNK // 2), CHUNK // 2)],
            ssems[h],
        )

    idx_load(0, 0).start()
    idx_load(0, 0).wait()

    def chunk(g, first):
        slot = lax.rem(g, 2)
        base = chunk_base(g)
        phase = base % PERIOD
        idx_load(g + 1, 1 - slot).start()
        for j in range(NSUB):
            for k in range(SUB // LANES):
                off = off_v[pl.ds(phase + j * SUB + k * LANES, LANES)]
                idx_v[slot, j, pl.ds(k * LANES, LANES)] = (
                    idx_v[slot, j, pl.ds(k * LANES, LANES)] + off
                )
        for h in (0, 1):
            @pl.when(jnp.logical_not(first))
            def _():
                make_store(base, h).wait()
            for j in range(HALF):
                pltpu.make_async_copy(
                    table_hbm.at[idx_v.at[slot, h * HALF + j]],
                    rows_v.at[h, pl.ds(j * SUB, SUB)],
                    gsems[h],
                ).start()
        for h in (0, 1):
            for j in range(HALF):
                pltpu.make_async_copy(
                    table_hbm.at[idx_v.at[slot, h * HALF + j]],
                    rows_v.at[h, pl.ds(j * SUB, SUB)],
                    gsems[h],
                ).wait()
            make_store(base, h).start()
        idx_load(g + 1, 1 - slot).wait()
        return jnp.bool_(False)

    lax.fori_loop(0, n_mine, chunk, jnp.bool_(True))
    for h in (0, 1):
        make_store(0, h).wait()


@jax.jit
def kernel(indices, tables):
    flat_tables = tables.reshape(F_ * V_, D_)
    idx2d = indices.reshape(N // SUB, SUB)
    off_lut = (jnp.arange(OFF_LEN, dtype=jnp.int32) // L_ % F_) * V_
    mesh = plsc.VectorSubcoreMesh(
        core_axis_name="c", subcore_axis_name="s", num_cores=NC, num_subcores=NS
    )
    out = pl.kernel(
        _body,
        out_type=jax.ShapeDtypeStruct((N, D_), jnp.float32),
        mesh=mesh,
        scratch_types=[
            pltpu.VMEM((2, NSUB, SUB), jnp.int32),
            pltpu.VMEM((2, CHUNK // 2, D_), jnp.float32),
            pltpu.VMEM((OFF_LEN,), jnp.int32),
            pltpu.SemaphoreType.DMA,
            pltpu.SemaphoreType.DMA,
            pltpu.SemaphoreType.DMA,
            pltpu.SemaphoreType.DMA,
            pltpu.SemaphoreType.DMA,
        ],
        compiler_params=pltpu.CompilerParams(use_tc_tiling_on_sc=False),
    )(idx2d, flat_tables, off_lut)
    return out.reshape(B, F_ * L_ * D_)


# speedup vs baseline: 1.2503x; 1.0156x over previous
import jax
import jax.numpy as jnp
from jax import lax
from jax.experimental import pallas as pl
from jax.experimental.pallas import tpu as pltpu
from jax.experimental.pallas import tpu_sc as plsc

NC, NS, LANES = 2, 16, 16
NW = NC * NS

B, F_, L_, V_, D_ = 1024, 26, 20, 1000, 64
N = B * F_ * L_
PERIOD = F_ * L_
SUB = 104
NSUB = PERIOD // SUB
SPC = 2
NCHUNK = B // SPC
NV16 = PERIOD // LANES


def _body(idx_hbm, table_hbm, off_hbm, out_hbm, idx_v, adj_v, rows_v, off_v,
          gsem0, gsem1, ssem0, ssem1, isem):
    wid = lax.axis_index("s") * NC + lax.axis_index("c")
    gsems = (gsem0, gsem1)
    ssems = (ssem0, ssem1)
    pltpu.sync_copy(off_hbm, off_v)
    n_mine = NCHUNK // NW

    def first_sample(g):
        return SPC * (g * NW + wid)

    def idx_load(g, slot):
        g = jnp.minimum(g, n_mine - 1)
        base = pl.multiple_of(first_sample(g) * PERIOD, SPC * PERIOD)
        return pltpu.make_async_copy(
            idx_hbm.at[pl.ds(base, SPC * PERIOD)], idx_v.at[slot], isem
        )

    def make_store(g, h):
        row0 = pl.multiple_of((first_sample(g) + h) * PERIOD, PERIOD)
        return pltpu.make_async_copy(
            rows_v.at[h],
            out_hbm.at[pl.ds(row0, PERIOD)],
            ssems[h],
        )

    idx_load(0, 0).start()
    idx_load(0, 0).wait()

    def chunk(g, first):
        slot = lax.rem(g, 2)
        idx_load(g + 1, 1 - slot).start()
        for h in range(SPC):
            hb = h * PERIOD
            starts = [k * LANES for k in range(NV16)] + [PERIOD - LANES]
            for s in starts:
                off = off_v[pl.ds(s, LANES)]
                adj_v[slot, pl.ds(hb + s, LANES)] = (
                    idx_v[slot, pl.ds(hb + s, LANES)] + off
                )
        for h in range(SPC):
            @pl.when(jnp.logical_not(first))
            def _():
                make_store(g, h).wait()
            for j in range(NSUB):
                pltpu.make_async_copy(
                    table_hbm.at[adj_v.at[slot, pl.ds(h * PERIOD + j * SUB, SUB)]],
                    rows_v.at[h, pl.ds(j * SUB, SUB)],
                    gsems[h],
                ).start()
        for h in range(SPC):
            for j in range(NSUB):
                pltpu.make_async_copy(
                    table_hbm.at[adj_v.at[slot, pl.ds(h * PERIOD + j * SUB, SUB)]],
                    rows_v.at[h, pl.ds(j * SUB, SUB)],
                    gsems[h],
                ).wait()
            make_store(g, h).start()
        idx_load(g + 1, 1 - slot).wait()
        return jnp.bool_(False)

    lax.fori_loop(0, n_mine, chunk, jnp.bool_(True))
    for h in range(SPC):
        make_store(0, h).wait()


@jax.jit
def kernel(indices, tables):
    flat_tables = tables.reshape(F_ * V_, D_)
    idx1d = indices.reshape(N)
    off_lut = (jnp.arange(PERIOD, dtype=jnp.int32) // L_) * V_
    mesh = plsc.VectorSubcoreMesh(
        core_axis_name="c", subcore_axis_name="s", num_cores=NC, num_subcores=NS
    )
    out = pl.kernel(
        _body,
        out_type=jax.ShapeDtypeStruct((N, D_), jnp.float32),
        mesh=mesh,
        scratch_types=[
            pltpu.VMEM((2, SPC * PERIOD), jnp.int32),
            pltpu.VMEM((2, SPC * PERIOD), jnp.int32),
            pltpu.VMEM((SPC, PERIOD, D_), jnp.float32),
            pltpu.VMEM((PERIOD,), jnp.int32),
            pltpu.SemaphoreType.DMA,
            pltpu.SemaphoreType.DMA,
            pltpu.SemaphoreType.DMA,
            pltpu.SemaphoreType.DMA,
            pltpu.SemaphoreType.DMA,
        ],
        compiler_params=pltpu.CompilerParams(use_tc_tiling_on_sc=False),
    )(idx1d, flat_tables, off_lut)
    return out.reshape(B, F_ * L_ * D_)
